# baseline (speedup 1.0000x reference)
import functools

import jax
import jax.numpy as jnp
from jax import lax
from jax.experimental import pallas as pl


@functools.lru_cache
def _make_tc_loss(B, C, bs):
    nblk = B // bs

    def body(cos_ref, phi_ref, lbl_ref, out_ref):
        i = pl.program_id(0)
        cos = cos_ref[...]
        phi = phi_ref[...]
        lbl = lbl_ref[...]
        mask = lax.broadcasted_iota(jnp.int32, (C, bs), 0) == lbl
        val = jnp.where(mask, phi, cos)
        psum = jnp.sum(jnp.where(mask, val, 0.0), keepdims=True)
        m = jnp.max(val, axis=0, keepdims=True)
        s = jnp.sum(jnp.exp(val - m), axis=0, keepdims=True)
        part = jnp.sum(m + jnp.log(s), keepdims=True) - psum

        @pl.when(i == 0)
        def _init():
            out_ref[...] = jnp.zeros_like(out_ref)

        out_ref[...] += part

        @pl.when(i == nblk - 1)
        def _final():
            out_ref[...] = out_ref[...] / B

    return pl.pallas_call(
        body,
        grid=(nblk,),
        in_specs=[
            pl.BlockSpec((C, bs), lambda i: (0, i)),
            pl.BlockSpec((C, bs), lambda i: (0, i)),
            pl.BlockSpec((1, bs), lambda i: (0, i)),
        ],
        out_specs=pl.BlockSpec((1, 1), lambda i: (0, 0)),
        out_shape=jax.ShapeDtypeStruct((1, 1), jnp.float32),
    )


def kernel(cos_theta, phi_theta, labels):
    B, C = cos_theta.shape
    out = _make_tc_loss(B, C, 1024)(
        cos_theta.T, phi_theta.T, labels.reshape(1, B)
    )
    return out[0, 0]


# speedup vs baseline: 1.0616x; 1.0616x over previous
import functools

import jax
import jax.numpy as jnp
from jax import lax
from jax.experimental import pallas as pl


@functools.lru_cache
def _make_tc_loss(B, C, bs):
    nblk = B // bs

    def body(cos_ref, phi_ref, lbl_ref, out_ref):
        i = pl.program_id(0)
        cos = cos_ref[...]
        phi = phi_ref[...]
        lbl = lbl_ref[...]
        mask = lax.broadcasted_iota(jnp.int32, (C, bs), 0) == lbl
        val = jnp.where(mask, phi, cos)
        p = jnp.sum(jnp.where(mask, phi, 0.0), axis=0, keepdims=True)
        m = jnp.max(val, axis=0, keepdims=True)
        s = jnp.sum(jnp.exp(val - m), axis=0, keepdims=True)
        part = jnp.sum(m + jnp.log(s) - p, keepdims=True)

        @pl.when(i == 0)
        def _init():
            out_ref[...] = jnp.zeros_like(out_ref)

        out_ref[...] += part

        @pl.when(i == nblk - 1)
        def _final():
            out_ref[...] = out_ref[...] / B

    return pl.pallas_call(
        body,
        grid=(nblk,),
        in_specs=[
            pl.BlockSpec((C, bs), lambda i: (0, i)),
            pl.BlockSpec((C, bs), lambda i: (0, i)),
            pl.BlockSpec((1, bs), lambda i: (0, i)),
        ],
        out_specs=pl.BlockSpec((1, 1), lambda i: (0, 0)),
        out_shape=jax.ShapeDtypeStruct((1, 1), jnp.float32),
    )


def kernel(cos_theta, phi_theta, labels):
    B, C = cos_theta.shape
    out = _make_tc_loss(B, C, 1024)(
        cos_theta.T, phi_theta.T, labels.reshape(1, B)
    )
    return out[0, 0]
